# Initial kernel scaffold; baseline (speedup 1.0000x reference)
import functools

import jax
import jax.numpy as jnp
from jax import lax
from jax.experimental import pallas as pl
from jax.experimental.pallas import tpu as pltpu
from jax.experimental.pallas import tpu_sc as plsc

N = 50000
E = 800000
H = 64
HH = 32
G = 64
CH = 128
EPAD = 802816
ROWS = EPAD // CH
CPT = ROWS // 16
CPW = ROWS // 32
ACC_ROWS = 50176
ZPT = ACC_ROWS // 16
NPT = N // 16
BLK = 1000
GRID = N // BLK

_MESH = plsc.VectorSubcoreMesh(core_axis_name="c", subcore_axis_name="s")


def _memset_rows(buf, rows, width, value):
  v = jnp.full((16,), value, jnp.float32)

  def row(i, _):
    for k in range(width // 16):
      buf[i, pl.ds(k * 16, 16)] = v
    return 0

  lax.fori_loop(0, rows, row, 0)


def _zero_acc_slice(acc, zbuf, base):
  for k in range(ZPT // CH):
    pltpu.sync_copy(zbuf, acc.at[pl.ds(base + k * CH, CH)])
  rem = ZPT - (ZPT // CH) * CH
  if rem:
    pltpu.sync_copy(zbuf.at[pl.ds(0, rem)],
                    acc.at[pl.ds(base + (ZPT // CH) * CH, rem)])


def _sc_deg_body(dst_i, dp0, dp1, dstb, obuf, dacc):
  c = lax.axis_index("c")
  s = lax.axis_index("s")
  wid = c * 16 + s
  _memset_rows(obuf, CH, 16, 0.0)
  _zero_acc_slice(dacc, obuf, s * ZPT)
  pltpu.sync_copy(dst_i.at[pl.ds(wid * CPW, CPW)], dstb)
  _memset_rows(obuf, CH, 16, 1.0)
  plsc.subcore_barrier()

  def body(j, _):
    pltpu.sync_copy(obuf, dacc.at[dstb.at[j]], add=True)
    return 0

  lax.fori_loop(0, CPW, body, 0)
  plsc.subcore_barrier()

  @pl.when(c == 0)
  def _():
    pltpu.sync_copy(dacc.at[pl.ds(s * NPT, NPT)], dp0.at[pl.ds(s * NPT, NPT)])

  @pl.when(c == 1)
  def _():
    pltpu.sync_copy(dacc.at[pl.ds(s * NPT, NPT)], dp1.at[pl.ds(s * NPT, NPT)])


_sc_deg = pl.kernel(
    _sc_deg_body,
    out_type=[jax.ShapeDtypeStruct((N, 16), jnp.float32)] * 2,
    mesh=_MESH,
    scratch_types=[
        pltpu.VMEM((CPW, CH), jnp.int32),
        pltpu.VMEM((CH, 16), jnp.float32),
        pltpu.VMEM_SHARED((ACC_ROWS, 16), jnp.float32),
    ],
)


def _sc_layer_body(h0, h1, src_i, dst_i, out0, out1, srcb, dstb, gbuf, acc, gsem):
  c = lax.axis_index("c")
  s = lax.axis_index("s")
  _memset_rows(gbuf.at[0], CH, HH, 0.0)
  _zero_acc_slice(acc, gbuf.at[0], s * ZPT)
  pltpu.sync_copy(src_i.at[pl.ds(s * CPT, CPT)], srcb)
  pltpu.sync_copy(dst_i.at[pl.ds(s * CPT, CPT)], dstb)
  plsc.subcore_barrier()

  def run(h_ref, out_ref):
    pltpu.async_copy(h_ref.at[srcb.at[0]], gbuf.at[0], gsem)

    def body(j, _):
      p = lax.rem(j, 2)
      q = lax.rem(j + 1, 2)
      pltpu.make_async_copy(h_ref.at[srcb.at[0]], gbuf.at[p], gsem).wait()
      pltpu.async_copy(h_ref.at[srcb.at[j + 1]], gbuf.at[q], gsem)
      pltpu.sync_copy(gbuf.at[p], acc.at[dstb.at[j]], add=True)
      return 0

    lax.fori_loop(0, CPT - 1, body, 0)
    p = (CPT - 1) % 2
    pltpu.make_async_copy(h_ref.at[srcb.at[0]], gbuf.at[p], gsem).wait()
    pltpu.sync_copy(gbuf.at[p], acc.at[dstb.at[CPT - 1]], add=True)
    plsc.subcore_barrier()
    pltpu.sync_copy(acc.at[pl.ds(s * NPT, NPT)], out_ref.at[pl.ds(s * NPT, NPT)])

  @pl.when(c == 0)
  def _():
    run(h0, out0)

  @pl.when(c == 1)
  def _():
    run(h1, out1)


_sc_layer = pl.kernel(
    _sc_layer_body,
    out_type=[jax.ShapeDtypeStruct((N, HH), jnp.float32)] * 2,
    mesh=_MESH,
    scratch_types=[
        pltpu.VMEM((CPT, CH), jnp.int32),
        pltpu.VMEM((CPT, CH), jnp.int32),
        pltpu.VMEM((2, CH, HH), jnp.float32),
        pltpu.VMEM_SHARED((ACC_ROWS, HH), jnp.float32),
        pltpu.SemaphoreType.DMA,
    ],
)


def _dinv_of(dp0_ref, dp1_ref):
  deg = 1.0 + dp0_ref[...][:, :1] + dp1_ref[...][:, :1]
  return lax.rsqrt(deg)


def _tc_a_body(x_ref, w_ref, dp0_ref, dp1_ref, h0_ref, h1_ref):
  dinv = _dinv_of(dp0_ref, dp1_ref)
  hp = jnp.dot(x_ref[...], w_ref[...], preferred_element_type=jnp.float32)
  hs = hp * dinv
  h0_ref[...] = hs[:, :HH]
  h1_ref[...] = hs[:, HH:]


def _tc_b_body(s0, s1, h0, h1, dp0, dp1, b_ref, w_ref, o0, o1):
  dinv = _dinv_of(dp0, dp1)
  z = jnp.concatenate([s0[...] + h0[...], s1[...] + h1[...]], axis=1)
  z = z * dinv + b_ref[...]
  a = jnp.maximum(z, 0.0)
  hn = jnp.dot(a, w_ref[...], preferred_element_type=jnp.float32) * dinv
  o0[...] = hn[:, :HH]
  o1[...] = hn[:, HH:]


def _tc_c_body(s0, s1, h0, h1, dp0, dp1, b_ref, batch_ref, h_out, pooled,
               sums, cnt):
  i = pl.program_id(0)
  dinv = _dinv_of(dp0, dp1)
  z = jnp.concatenate([s0[...] + h0[...], s1[...] + h1[...]], axis=1)
  z = z * dinv + b_ref[...]
  h_out[...] = z
  bt = batch_ref[0]
  iota = lax.broadcasted_iota(jnp.int32, (G, BLK), 0)
  msk = (iota == bt).astype(jnp.float32)

  @pl.when(i == 0)
  def _():
    sums[...] = jnp.zeros_like(sums)
    cnt[...] = jnp.zeros_like(cnt)

  sums[...] += jnp.dot(msk, z, preferred_element_type=jnp.float32)
  cnt[...] += jnp.dot(msk, jnp.ones((BLK, 1), jnp.float32),
                      preferred_element_type=jnp.float32)
  pooled[...] = sums[...] / jnp.maximum(cnt[...], 1.0)


def _row_spec(w):
  return pl.BlockSpec((BLK, w), lambda i: (i, 0))


def _full_spec(shape):
  return pl.BlockSpec(shape, lambda i: tuple(0 for _ in shape))


_tc_a = pl.pallas_call(
    _tc_a_body,
    grid=(GRID,),
    in_specs=[_row_spec(8), _full_spec((8, H)), _row_spec(16), _row_spec(16)],
    out_specs=[_row_spec(HH), _row_spec(HH)],
    out_shape=[jax.ShapeDtypeStruct((N, HH), jnp.float32)] * 2,
)

_tc_b = pl.pallas_call(
    _tc_b_body,
    grid=(GRID,),
    in_specs=[_row_spec(HH), _row_spec(HH), _row_spec(HH), _row_spec(HH),
              _row_spec(16), _row_spec(16), _full_spec((1, H)),
              _full_spec((H, H))],
    out_specs=[_row_spec(HH), _row_spec(HH)],
    out_shape=[jax.ShapeDtypeStruct((N, HH), jnp.float32)] * 2,
)

_tc_c = pl.pallas_call(
    _tc_c_body,
    grid=(GRID,),
    in_specs=[_row_spec(HH), _row_spec(HH), _row_spec(HH), _row_spec(HH),
              _row_spec(16), _row_spec(16), _full_spec((1, H)),
              pl.BlockSpec((1, 1, BLK), lambda i: (i, 0, 0))],
    out_specs=[_row_spec(H), _full_spec((G, H))],
    out_shape=[jax.ShapeDtypeStruct((N, H), jnp.float32),
               jax.ShapeDtypeStruct((G, H), jnp.float32)],
    scratch_shapes=[pltpu.VMEM((G, H), jnp.float32),
                    pltpu.VMEM((G, 1), jnp.float32)],
)


@jax.jit
def kernel(x, edge_index, batch, W1, b1, W2, b2, W3, b3):
  src = edge_index[0]
  dst = edge_index[1]
  pad = EPAD - E
  src_i = jnp.concatenate([src, jnp.zeros((pad,), jnp.int32)]).reshape(ROWS, CH)
  dst_i = jnp.concatenate([dst, jnp.full((pad,), N, jnp.int32)]).reshape(ROWS, CH)
  xp = jnp.pad(x, ((0, 0), (0, 1)))
  W1p = jnp.pad(W1, ((0, 1), (0, 0)))

  dp0, dp1 = _sc_deg(dst_i)
  h0, h1 = _tc_a(xp, W1p, dp0, dp1)
  s0, s1 = _sc_layer(h0, h1, src_i, dst_i)
  h0, h1 = _tc_b(s0, s1, h0, h1, dp0, dp1, b1.reshape(1, H), W2)
  s0, s1 = _sc_layer(h0, h1, src_i, dst_i)
  h0, h1 = _tc_b(s0, s1, h0, h1, dp0, dp1, b2.reshape(1, H), W3)
  s0, s1 = _sc_layer(h0, h1, src_i, dst_i)
  h, pooled = _tc_c(s0, s1, h0, h1, dp0, dp1, b3.reshape(1, H),
                    batch.reshape(GRID, 1, BLK))
  return (h, pooled)


# speedup vs baseline: 6.5581x; 6.5581x over previous
import jax
import jax.numpy as jnp
from jax import lax
from jax.experimental import pallas as pl
from jax.experimental.pallas import tpu as pltpu
from jax.experimental.pallas import tpu_sc as plsc

N = 50000
E = 800000
H = 64
HQ = 16
G = 64
CH = 128
EPAD = 819200
ROWS = EPAD // CH
CPT = ROWS // 16
KB = 80
NB = CPT // KB
ACC_ROWS = 50176
ZPT = ACC_ROWS // 16
BLK = 1000
GRID = N // BLK

_MESH = plsc.VectorSubcoreMesh(core_axis_name="c", subcore_axis_name="s")
_SC_PARAMS = pltpu.CompilerParams(use_tc_tiling_on_sc=False)


def _memset_rows(buf, rows, value):
  v = jnp.full((16,), value, jnp.float32)

  def row(i, _):
    buf[i, pl.ds(0, 16)] = v
    return 0

  lax.fori_loop(0, rows, row, 0)


def _zero_acc_slice(acc, zbuf, base):
  for k in range(ZPT // CH):
    pltpu.sync_copy(zbuf, acc.at[pl.ds(base + k * CH, CH)])
  rem = ZPT - (ZPT // CH) * CH
  if rem:
    pltpu.sync_copy(zbuf.at[pl.ds(0, rem)],
                    acc.at[pl.ds(base + (ZPT // CH) * CH, rem)])


def _sc_layer_body(h0, h1, h2, h3, src_i, dst_i, o0, o1, o2, o3,
                   srcb, dstb, gbuf, acc, gsem):
  c = lax.axis_index("c")
  s = lax.axis_index("s")

  def run(h_ref, out_ref):
    _memset_rows(gbuf.at[0], CH, 0.0)
    _zero_acc_slice(acc, gbuf.at[0], s * ZPT)
    plsc.subcore_barrier()

    for blk in range(NB):
      base = s * CPT + blk * KB
      pltpu.sync_copy(src_i.at[pl.ds(base, KB)], srcb)
      pltpu.sync_copy(dst_i.at[pl.ds(base, KB)], dstb)
      pltpu.async_copy(h_ref.at[srcb.at[0]], gbuf.at[0], gsem)

      def body(j, _):
        p = lax.rem(j, 2)
        q = lax.rem(j + 1, 2)
        pltpu.make_async_copy(h_ref.at[srcb.at[0]], gbuf.at[p], gsem).wait()
        pltpu.async_copy(h_ref.at[srcb.at[j + 1]], gbuf.at[q], gsem)
        pltpu.sync_copy(gbuf.at[p], acc.at[dstb.at[j]], add=True)
        return 0

      lax.fori_loop(0, KB - 1, body, 0)
      p = (KB - 1) % 2
      pltpu.make_async_copy(h_ref.at[srcb.at[0]], gbuf.at[p], gsem).wait()
      pltpu.sync_copy(gbuf.at[p], acc.at[dstb.at[KB - 1]], add=True)

    plsc.subcore_barrier()
    pltpu.sync_copy(acc.at[pl.ds(s * ZPT, ZPT)], out_ref.at[pl.ds(s * ZPT, ZPT)])
    plsc.subcore_barrier()

  @pl.when(c == 0)
  def _():
    run(h0, o0)
    run(h1, o1)

  @pl.when(c == 1)
  def _():
    run(h2, o2)
    run(h3, o3)


_sc_layer = pl.kernel(
    _sc_layer_body,
    out_type=[jax.ShapeDtypeStruct((ACC_ROWS, HQ), jnp.float32)] * 4,
    mesh=_MESH,
    scratch_types=[
        pltpu.VMEM((KB, CH), jnp.int32),
        pltpu.VMEM((KB, CH), jnp.int32),
        pltpu.VMEM((2, CH, HQ), jnp.float32),
        pltpu.VMEM_SHARED((ACC_ROWS, HQ), jnp.float32),
        pltpu.SemaphoreType.DMA,
    ],
    compiler_params=_SC_PARAMS,
)


def _split4(ref_list, val):
  for q, r in enumerate(ref_list):
    r[...] = val[:, q * HQ:(q + 1) * HQ]


def _tc_mid_body(s0, s1, s2, s3, c0, c1, c2, c3, dpc, x_ref, w1_ref,
                 wn_ref, b_ref, ff_ref, lf_ref, o0, o1, o2, o3, dpo):
  ff = ff_ref[...] > 0.0
  lf = lf_ref[...] > 0.0
  deg_cols = jnp.where(ff, s0[...], dpc[...])
  dpo[...] = deg_cols
  dinv = lax.rsqrt(1.0 + deg_cols[:, :1])
  hp = jnp.dot(x_ref[...], w1_ref[...],
               preferred_element_type=jnp.float32) * dinv
  z = jnp.concatenate(
      [s[...] + c[...] for s, c in zip((s0, s1, s2, s3), (c0, c1, c2, c3))],
      axis=1)
  z = z * dinv + b_ref[...]
  a = jnp.maximum(z, 0.0)
  y = jnp.dot(a, wn_ref[...], preferred_element_type=jnp.float32) * dinv
  out = jnp.where(ff, hp, jnp.where(lf, z, y))
  _split4([o0, o1, o2, o3], out)


def _tc_pool_body(h0, h1, h2, h3, batch_ref, h_out, pooled, sums, cnt):
  i = pl.program_id(0)
  z = jnp.concatenate([h0[...], h1[...], h2[...], h3[...]], axis=1)
  h_out[...] = z
  bt = batch_ref[0]
  iota = lax.broadcasted_iota(jnp.int32, (G, BLK), 0)
  msk = (iota == bt).astype(jnp.float32)

  @pl.when(i == 0)
  def _():
    sums[...] = jnp.zeros_like(sums)
    cnt[...] = jnp.zeros_like(cnt)

  sums[...] += jnp.dot(msk, z, preferred_element_type=jnp.float32)
  cnt[...] += jnp.dot(msk, jnp.ones((BLK, 1), jnp.float32),
                      preferred_element_type=jnp.float32)
  pooled[...] = sums[...] / jnp.maximum(cnt[...], 1.0)


def _row_spec(w):
  return pl.BlockSpec((BLK, w), lambda i: (i, 0))


def _full_spec(shape):
  return pl.BlockSpec(shape, lambda i: tuple(0 for _ in shape))


_Q = [jax.ShapeDtypeStruct((N, HQ), jnp.float32)] * 4

_tc_mid = pl.pallas_call(
    _tc_mid_body,
    grid=(GRID,),
    in_specs=[_row_spec(HQ)] * 8 + [_row_spec(16), _row_spec(8),
              _full_spec((8, H)), _full_spec((H, H)), _full_spec((1, H)),
              _full_spec((1, 1)), _full_spec((1, 1))],
    out_specs=[_row_spec(HQ)] * 5,
    out_shape=_Q + [jax.ShapeDtypeStruct((N, 16), jnp.float32)],
)

_tc_pool = pl.pallas_call(
    _tc_pool_body,
    grid=(GRID,),
    in_specs=[_row_spec(HQ)] * 4 +
             [pl.BlockSpec((1, 1, BLK), lambda i: (i, 0, 0))],
    out_specs=[_row_spec(H), _full_spec((G, H))],
    out_shape=[jax.ShapeDtypeStruct((N, H), jnp.float32),
               jax.ShapeDtypeStruct((G, H), jnp.float32)],
    scratch_shapes=[pltpu.VMEM((G, H), jnp.float32),
                    pltpu.VMEM((G, 1), jnp.float32)],
)


@jax.jit
def kernel(x, edge_index, batch, W1, b1, W2, b2, W3, b3):
  src = edge_index[0]
  dst = edge_index[1]
  pad = EPAD - E
  src_i = jnp.concatenate([src, jnp.zeros((pad,), jnp.int32)]).reshape(ROWS, CH)
  dst_i = jnp.concatenate([dst, jnp.full((pad,), N, jnp.int32)]).reshape(ROWS, CH)
  xp = jnp.pad(x, ((0, 0), (0, 1)))
  W1p = jnp.pad(W1, ((0, 1), (0, 0)))

  ones_q = jnp.ones((N, HQ), jnp.float32)
  carry = (ones_q, ones_q, ones_q, ones_q, jnp.zeros((N, 16), jnp.float32))

  zH = jnp.zeros((H, H), jnp.float32)
  z1 = jnp.zeros((1, H), jnp.float32)
  wn_stack = jnp.stack([zH, W2, W3, zH])
  b_stack = jnp.stack([z1, b1.reshape(1, H), b2.reshape(1, H),
                       b3.reshape(1, H)])
  ff = jnp.array([1.0, 0.0, 0.0, 0.0], jnp.float32).reshape(4, 1, 1)
  lf = jnp.array([0.0, 0.0, 0.0, 1.0], jnp.float32).reshape(4, 1, 1)

  def body(carry, xs):
    wn, b, f0, f1 = xs
    sq = _sc_layer(*carry[:4], src_i, dst_i)
    nxt = _tc_mid(*sq, *carry, xp, W1p, wn, b, f0, f1)
    return tuple(nxt), None

  carry, _ = lax.scan(body, carry, (wn_stack, b_stack, ff, lf))
  h, pooled = _tc_pool(*carry[:4], batch.reshape(GRID, 1, BLK))
  return (h, pooled)


# speedup vs baseline: 8.9065x; 1.3581x over previous
import jax
import jax.numpy as jnp
from jax import lax
from jax.experimental import pallas as pl
from jax.experimental.pallas import tpu as pltpu
from jax.experimental.pallas import tpu_sc as plsc

N = 50000
E = 800000
H = 64
HQ = 16
G = 64
CH = 128
EPAD = 819200
ROWS = EPAD // CH
CPT = ROWS // 16
KB = 80
NB = CPT // KB
NBUF = 8
PG = 4
PS = NBUF - PG
ACC_ROWS = 50176
ZPT = ACC_ROWS // 16
BLK = 1000
GRID = N // BLK

_MESH = plsc.VectorSubcoreMesh(core_axis_name="c", subcore_axis_name="s")
_SC_PARAMS = pltpu.CompilerParams(use_tc_tiling_on_sc=False)


def _memset_rows(buf, rows, value):
  v = jnp.full((16,), value, jnp.float32)

  def row(i, _):
    buf[i, pl.ds(0, 16)] = v
    return 0

  lax.fori_loop(0, rows, row, 0)


def _zero_acc_slice(acc, zbuf, base):
  for k in range(ZPT // CH):
    pltpu.sync_copy(zbuf, acc.at[pl.ds(base + k * CH, CH)])
  rem = ZPT - (ZPT // CH) * CH
  if rem:
    pltpu.sync_copy(zbuf.at[pl.ds(0, rem)],
                    acc.at[pl.ds(base + (ZPT // CH) * CH, rem)])


def _sc_layer_body(h0, h1, h2, h3, src_i, dst_i, o0, o1, o2, o3,
                   srcb, dstb, gbuf, acc, gsem, ssem):
  c = lax.axis_index("c")
  s = lax.axis_index("s")

  def run(h_ref, out_ref):
    _memset_rows(gbuf.at[0], CH, 0.0)
    _zero_acc_slice(acc, gbuf.at[0], s * ZPT)
    plsc.subcore_barrier()

    def gather(k):
      pltpu.async_copy(h_ref.at[srcb.at[k]], gbuf.at[lax.rem(k, NBUF)], gsem)

    def scatter(k):
      pltpu.async_copy(gbuf.at[lax.rem(k, NBUF)], acc.at[dstb.at[k]], ssem,
                       add=True)

    def wait_g():
      pltpu.make_async_copy(h_ref.at[srcb.at[0]], gbuf.at[0], gsem).wait()

    def wait_s():
      pltpu.make_async_copy(gbuf.at[0], acc.at[dstb.at[0]], ssem).wait()

    for blk in range(NB):
      base = s * CPT + blk * KB
      pltpu.sync_copy(src_i.at[pl.ds(base, KB)], srcb)
      pltpu.sync_copy(dst_i.at[pl.ds(base, KB)], dstb)

      for k in range(PG):
        gather(k)

      def body1(k, _):
        wait_g()
        scatter(k)
        gather(k + PG)
        return 0

      def body2(k, _):
        wait_g()
        scatter(k)
        wait_s()
        gather(k + PG)
        return 0

      def body3(k, _):
        wait_g()
        scatter(k)
        wait_s()
        return 0

      lax.fori_loop(0, PS, body1, 0)
      lax.fori_loop(PS, KB - PG, body2, 0)
      lax.fori_loop(KB - PG, KB, body3, 0)
      for _ in range(PS):
        wait_s()

    plsc.subcore_barrier()
    pltpu.sync_copy(acc.at[pl.ds(s * ZPT, ZPT)], out_ref.at[pl.ds(s * ZPT, ZPT)])
    plsc.subcore_barrier()

  @pl.when(c == 0)
  def _():
    run(h0, o0)
    run(h1, o1)

  @pl.when(c == 1)
  def _():
    run(h2, o2)
    run(h3, o3)


_sc_layer = pl.kernel(
    _sc_layer_body,
    out_type=[jax.ShapeDtypeStruct((ACC_ROWS, HQ), jnp.float32)] * 4,
    mesh=_MESH,
    scratch_types=[
        pltpu.VMEM((KB, CH), jnp.int32),
        pltpu.VMEM((KB, CH), jnp.int32),
        pltpu.VMEM((NBUF, CH, HQ), jnp.float32),
        pltpu.VMEM_SHARED((ACC_ROWS, HQ), jnp.float32),
        pltpu.SemaphoreType.DMA,
        pltpu.SemaphoreType.DMA,
    ],
    compiler_params=_SC_PARAMS,
)


def _split4(ref_list, val):
  for q, r in enumerate(ref_list):
    r[...] = val[:, q * HQ:(q + 1) * HQ]


def _tc_mid_body(s0, s1, s2, s3, c0, c1, c2, c3, dpc, x_ref, w1_ref,
                 wn_ref, b_ref, ff_ref, lf_ref, o0, o1, o2, o3, dpo):
  ff = ff_ref[...] > 0.0
  lf = lf_ref[...] > 0.0
  deg_cols = jnp.where(ff, s0[...], dpc[...])
  dpo[...] = deg_cols
  dinv = lax.rsqrt(1.0 + deg_cols[:, :1])
  hp = jnp.dot(x_ref[...], w1_ref[...],
               preferred_element_type=jnp.float32) * dinv
  z = jnp.concatenate(
      [s[...] + c[...] for s, c in zip((s0, s1, s2, s3), (c0, c1, c2, c3))],
      axis=1)
  z = z * dinv + b_ref[...]
  a = jnp.maximum(z, 0.0)
  y = jnp.dot(a, wn_ref[...], preferred_element_type=jnp.float32) * dinv
  out = jnp.where(ff, hp, jnp.where(lf, z, y))
  _split4([o0, o1, o2, o3], out)


def _tc_pool_body(h0, h1, h2, h3, batch_ref, h_out, pooled, sums, cnt):
  i = pl.program_id(0)
  z = jnp.concatenate([h0[...], h1[...], h2[...], h3[...]], axis=1)
  h_out[...] = z
  bt = batch_ref[0]
  iota = lax.broadcasted_iota(jnp.int32, (G, BLK), 0)
  msk = (iota == bt).astype(jnp.float32)

  @pl.when(i == 0)
  def _():
    sums[...] = jnp.zeros_like(sums)
    cnt[...] = jnp.zeros_like(cnt)

  sums[...] += jnp.dot(msk, z, preferred_element_type=jnp.float32)
  cnt[...] += jnp.dot(msk, jnp.ones((BLK, 1), jnp.float32),
                      preferred_element_type=jnp.float32)
  pooled[...] = sums[...] / jnp.maximum(cnt[...], 1.0)


def _row_spec(w):
  return pl.BlockSpec((BLK, w), lambda i: (i, 0))


def _full_spec(shape):
  return pl.BlockSpec(shape, lambda i: tuple(0 for _ in shape))


_Q = [jax.ShapeDtypeStruct((N, HQ), jnp.float32)] * 4

_tc_mid = pl.pallas_call(
    _tc_mid_body,
    grid=(GRID,),
    in_specs=[_row_spec(HQ)] * 8 + [_row_spec(16), _row_spec(8),
              _full_spec((8, H)), _full_spec((H, H)), _full_spec((1, H)),
              _full_spec((1, 1)), _full_spec((1, 1))],
    out_specs=[_row_spec(HQ)] * 5,
    out_shape=_Q + [jax.ShapeDtypeStruct((N, 16), jnp.float32)],
)

_tc_pool = pl.pallas_call(
    _tc_pool_body,
    grid=(GRID,),
    in_specs=[_row_spec(HQ)] * 4 +
             [pl.BlockSpec((1, 1, BLK), lambda i: (i, 0, 0))],
    out_specs=[_row_spec(H), _full_spec((G, H))],
    out_shape=[jax.ShapeDtypeStruct((N, H), jnp.float32),
               jax.ShapeDtypeStruct((G, H), jnp.float32)],
    scratch_shapes=[pltpu.VMEM((G, H), jnp.float32),
                    pltpu.VMEM((G, 1), jnp.float32)],
)


@jax.jit
def kernel(x, edge_index, batch, W1, b1, W2, b2, W3, b3):
  src = edge_index[0]
  dst = edge_index[1]
  pad = EPAD - E
  src_i = jnp.concatenate([src, jnp.zeros((pad,), jnp.int32)]).reshape(ROWS, CH)
  dst_i = jnp.concatenate([dst, jnp.full((pad,), N, jnp.int32)]).reshape(ROWS, CH)
  xp = jnp.pad(x, ((0, 0), (0, 1)))
  W1p = jnp.pad(W1, ((0, 1), (0, 0)))

  ones_q = jnp.ones((N, HQ), jnp.float32)
  carry = (ones_q, ones_q, ones_q, ones_q, jnp.zeros((N, 16), jnp.float32))

  zH = jnp.zeros((H, H), jnp.float32)
  z1 = jnp.zeros((1, H), jnp.float32)
  wn_stack = jnp.stack([zH, W2, W3, zH])
  b_stack = jnp.stack([z1, b1.reshape(1, H), b2.reshape(1, H),
                       b3.reshape(1, H)])
  ff = jnp.array([1.0, 0.0, 0.0, 0.0], jnp.float32).reshape(4, 1, 1)
  lf = jnp.array([0.0, 0.0, 0.0, 1.0], jnp.float32).reshape(4, 1, 1)

  def body(carry, xs):
    wn, b, f0, f1 = xs
    sq = _sc_layer(*carry[:4], src_i, dst_i)
    nxt = _tc_mid(*sq, *carry, xp, W1p, wn, b, f0, f1)
    return tuple(nxt), None

  carry, _ = lax.scan(body, carry, (wn_stack, b_stack, ff, lf))
  h, pooled = _tc_pool(*carry[:4], batch.reshape(GRID, 1, BLK))
  return (h, pooled)


# speedup vs baseline: 9.0024x; 1.0108x over previous
import jax
import jax.numpy as jnp
from jax import lax
from jax.experimental import pallas as pl
from jax.experimental.pallas import tpu as pltpu
from jax.experimental.pallas import tpu_sc as plsc

N = 50000
E = 800000
H = 64
HQ = 16
G = 64
CH = 128
EPAD = 819200
ROWS = EPAD // CH
CPT = ROWS // 16
KB = 200
NB = CPT // KB
NBUF = 8
PG = 4
PS = NBUF - PG
ACC_ROWS = 50176
ZPT = ACC_ROWS // 16
BLK = 1000
GRID = N // BLK

_MESH = plsc.VectorSubcoreMesh(core_axis_name="c", subcore_axis_name="s")
_SC_PARAMS = pltpu.CompilerParams(use_tc_tiling_on_sc=False)


def _memset_rows(buf, rows, value):
  v = jnp.full((16,), value, jnp.float32)

  def row(i, _):
    buf[i, pl.ds(0, 16)] = v
    return 0

  lax.fori_loop(0, rows, row, 0)


def _zero_acc_slice(acc, zbuf, base):
  for k in range(ZPT // CH):
    pltpu.sync_copy(zbuf, acc.at[pl.ds(base + k * CH, CH)])
  rem = ZPT - (ZPT // CH) * CH
  if rem:
    pltpu.sync_copy(zbuf.at[pl.ds(0, rem)],
                    acc.at[pl.ds(base + (ZPT // CH) * CH, rem)])


def _sc_layer_body(h0, h1, h2, h3, src_i, dst_i, o0, o1, o2, o3,
                   srcb, dstb, gbuf, acc, gsem, ssem):
  c = lax.axis_index("c")
  s = lax.axis_index("s")

  def run(h_ref, out_ref):
    _memset_rows(gbuf.at[0], CH, 0.0)
    _zero_acc_slice(acc, gbuf.at[0], s * ZPT)
    plsc.subcore_barrier()

    def gather(k):
      pltpu.async_copy(h_ref.at[srcb.at[k]], gbuf.at[lax.rem(k, NBUF)], gsem)

    def scatter(k):
      pltpu.async_copy(gbuf.at[lax.rem(k, NBUF)], acc.at[dstb.at[k]], ssem,
                       add=True)

    def wait_g():
      pltpu.make_async_copy(h_ref.at[srcb.at[0]], gbuf.at[0], gsem).wait()

    def wait_s():
      pltpu.make_async_copy(gbuf.at[0], acc.at[dstb.at[0]], ssem).wait()

    for blk in range(NB):
      base = s * CPT + blk * KB
      pltpu.sync_copy(src_i.at[pl.ds(base, KB)], srcb)
      pltpu.sync_copy(dst_i.at[pl.ds(base, KB)], dstb)

      for k in range(PG):
        gather(k)

      def body1(k, _):
        wait_g()
        scatter(k)
        gather(k + PG)
        return 0

      def body2(k, _):
        wait_g()
        scatter(k)
        wait_s()
        gather(k + PG)
        return 0

      def body3(k, _):
        wait_g()
        scatter(k)
        wait_s()
        return 0

      lax.fori_loop(0, PS, body1, 0)
      lax.fori_loop(PS, KB - PG, body2, 0, unroll=2)
      lax.fori_loop(KB - PG, KB, body3, 0)
      for _ in range(PS):
        wait_s()

    plsc.subcore_barrier()
    pltpu.sync_copy(acc.at[pl.ds(s * ZPT, ZPT)], out_ref.at[pl.ds(s * ZPT, ZPT)])
    plsc.subcore_barrier()

  @pl.when(c == 0)
  def _():
    run(h0, o0)
    run(h1, o1)

  @pl.when(c == 1)
  def _():
    run(h2, o2)
    run(h3, o3)


_sc_layer = pl.kernel(
    _sc_layer_body,
    out_type=[jax.ShapeDtypeStruct((ACC_ROWS, HQ), jnp.float32)] * 4,
    mesh=_MESH,
    scratch_types=[
        pltpu.VMEM((KB, CH), jnp.int32),
        pltpu.VMEM((KB, CH), jnp.int32),
        pltpu.VMEM((NBUF, CH, HQ), jnp.float32),
        pltpu.VMEM_SHARED((ACC_ROWS, HQ), jnp.float32),
        pltpu.SemaphoreType.DMA,
        pltpu.SemaphoreType.DMA,
    ],
    compiler_params=_SC_PARAMS,
)


def _split4(ref_list, val):
  for q, r in enumerate(ref_list):
    r[...] = val[:, q * HQ:(q + 1) * HQ]


def _tc_mid_body(s0, s1, s2, s3, c0, c1, c2, c3, dpc, x_ref, w1_ref,
                 wn_ref, b_ref, ff_ref, lf_ref, o0, o1, o2, o3, dpo):
  ff = ff_ref[...] > 0.0
  lf = lf_ref[...] > 0.0
  deg_cols = jnp.where(ff, s0[...], dpc[...])
  dpo[...] = deg_cols
  dinv = lax.rsqrt(1.0 + deg_cols[:, :1])
  hp = jnp.dot(x_ref[...], w1_ref[...],
               preferred_element_type=jnp.float32) * dinv
  z = jnp.concatenate(
      [s[...] + c[...] for s, c in zip((s0, s1, s2, s3), (c0, c1, c2, c3))],
      axis=1)
  z = z * dinv + b_ref[...]
  a = jnp.maximum(z, 0.0)
  y = jnp.dot(a, wn_ref[...], preferred_element_type=jnp.float32) * dinv
  out = jnp.where(ff, hp, jnp.where(lf, z, y))
  _split4([o0, o1, o2, o3], out)


def _tc_pool_body(h0, h1, h2, h3, batch_ref, h_out, pooled, sums, cnt):
  i = pl.program_id(0)
  z = jnp.concatenate([h0[...], h1[...], h2[...], h3[...]], axis=1)
  h_out[...] = z
  bt = batch_ref[0]
  iota = lax.broadcasted_iota(jnp.int32, (G, BLK), 0)
  msk = (iota == bt).astype(jnp.float32)

  @pl.when(i == 0)
  def _():
    sums[...] = jnp.zeros_like(sums)
    cnt[...] = jnp.zeros_like(cnt)

  sums[...] += jnp.dot(msk, z, preferred_element_type=jnp.float32)
  cnt[...] += jnp.dot(msk, jnp.ones((BLK, 1), jnp.float32),
                      preferred_element_type=jnp.float32)
  pooled[...] = sums[...] / jnp.maximum(cnt[...], 1.0)


def _row_spec(w):
  return pl.BlockSpec((BLK, w), lambda i: (i, 0))


def _full_spec(shape):
  return pl.BlockSpec(shape, lambda i: tuple(0 for _ in shape))


_Q = [jax.ShapeDtypeStruct((N, HQ), jnp.float32)] * 4

_tc_mid = pl.pallas_call(
    _tc_mid_body,
    grid=(GRID,),
    in_specs=[_row_spec(HQ)] * 8 + [_row_spec(16), _row_spec(8),
              _full_spec((8, H)), _full_spec((H, H)), _full_spec((1, H)),
              _full_spec((1, 1)), _full_spec((1, 1))],
    out_specs=[_row_spec(HQ)] * 5,
    out_shape=_Q + [jax.ShapeDtypeStruct((N, 16), jnp.float32)],
)

_tc_pool = pl.pallas_call(
    _tc_pool_body,
    grid=(GRID,),
    in_specs=[_row_spec(HQ)] * 4 +
             [pl.BlockSpec((1, 1, BLK), lambda i: (i, 0, 0))],
    out_specs=[_row_spec(H), _full_spec((G, H))],
    out_shape=[jax.ShapeDtypeStruct((N, H), jnp.float32),
               jax.ShapeDtypeStruct((G, H), jnp.float32)],
    scratch_shapes=[pltpu.VMEM((G, H), jnp.float32),
                    pltpu.VMEM((G, 1), jnp.float32)],
)


@jax.jit
def kernel(x, edge_index, batch, W1, b1, W2, b2, W3, b3):
  src = edge_index[0]
  dst = edge_index[1]
  pad = EPAD - E
  src_i = jnp.concatenate([src, jnp.zeros((pad,), jnp.int32)]).reshape(ROWS, CH)
  dst_i = jnp.concatenate([dst, jnp.full((pad,), N, jnp.int32)]).reshape(ROWS, CH)
  xp = jnp.pad(x, ((0, 0), (0, 1)))
  W1p = jnp.pad(W1, ((0, 1), (0, 0)))

  ones_q = jnp.ones((N, HQ), jnp.float32)
  carry = (ones_q, ones_q, ones_q, ones_q, jnp.zeros((N, 16), jnp.float32))

  zH = jnp.zeros((H, H), jnp.float32)
  z1 = jnp.zeros((1, H), jnp.float32)
  wn_stack = jnp.stack([zH, W2, W3, zH])
  b_stack = jnp.stack([z1, b1.reshape(1, H), b2.reshape(1, H),
                       b3.reshape(1, H)])
  ff = jnp.array([1.0, 0.0, 0.0, 0.0], jnp.float32).reshape(4, 1, 1)
  lf = jnp.array([0.0, 0.0, 0.0, 1.0], jnp.float32).reshape(4, 1, 1)

  def body(carry, xs):
    wn, b, f0, f1 = xs
    sq = _sc_layer(*carry[:4], src_i, dst_i)
    nxt = _tc_mid(*sq, *carry, xp, W1p, wn, b, f0, f1)
    return tuple(nxt), None

  carry, _ = lax.scan(body, carry, (wn_stack, b_stack, ff, lf))
  h, pooled = _tc_pool(*carry[:4], batch.reshape(GRID, 1, BLK))
  return (h, pooled)
